# baseline (speedup 1.0000x reference)
import functools

import jax
import jax.numpy as jnp
from jax import lax
from jax.experimental import pallas as pl
from jax.experimental.pallas import tpu as pltpu
from jax.experimental.pallas import tpu_sc as plsc

N = 8388608
NW = 32
C = N // NW
CH = 8192
NCHUNK = C // CH
VREGS = CH // 16

TBL = 196

ROWS = N // 128
R = 1024
G = ROWS // R


def _pop6(x):
    return (((x >> 0) & 1) + ((x >> 1) & 1) + ((x >> 2) & 1)
            + ((x >> 3) & 1) + ((x >> 4) & 1) + ((x >> 5) & 1))


def _sc_hist_body(q_hbm, mic_hbm, fam_hbm, om_hbm, eff_hbm, out_hbm,
                  qb, mb, fb, ob, vb, tbl, sem):
    wid = lax.axis_index("s") * 2 + lax.axis_index("c")
    base = wid * C
    lane = lax.iota(jnp.int32, 16)
    zeros16 = jnp.zeros((16,), jnp.int32)
    gidx = lane >> 2
    shv = (lane & 3) * 8

    def zbody(i, _):
        tbl[pl.ds(i * 16, 16)] = zeros16
        return 0
    lax.fori_loop(0, TBL, zbody, 0)

    def issue(c, p):
        off = base + c * CH
        offw = pl.multiple_of((base + c * CH) // 4, 2048)
        pltpu.async_copy(q_hbm.at[pl.ds(off, CH)], qb.at[p], sem.at[p])
        pltpu.async_copy(mic_hbm.at[pl.ds(off, CH)], mb.at[p], sem.at[p])
        pltpu.async_copy(fam_hbm.at[pl.ds(off, CH)], fb.at[p], sem.at[p])
        pltpu.async_copy(om_hbm.at[pl.ds(off, CH)], ob.at[p], sem.at[p])
        pltpu.async_copy(eff_hbm.at[pl.ds(offw, CH // 4)], vb.at[p], sem.at[p])

    def drain(c, p):
        off = base + c * CH
        offw = pl.multiple_of((base + c * CH) // 4, 2048)
        pltpu.make_async_copy(q_hbm.at[pl.ds(off, CH)], qb.at[p], sem.at[p]).wait()
        pltpu.make_async_copy(mic_hbm.at[pl.ds(off, CH)], mb.at[p], sem.at[p]).wait()
        pltpu.make_async_copy(fam_hbm.at[pl.ds(off, CH)], fb.at[p], sem.at[p]).wait()
        pltpu.make_async_copy(om_hbm.at[pl.ds(off, CH)], ob.at[p], sem.at[p]).wait()
        pltpu.make_async_copy(eff_hbm.at[pl.ds(offw, CH // 4)], vb.at[p], sem.at[p]).wait()

    issue(0, 0)

    def chunk_body(c, _):
        p = c & 1

        @pl.when(c + 1 < NCHUNK)
        def _():
            issue(c + 1, 1 - p)

        drain(c, p)

        @plsc.parallel_loop(0, VREGS // 4, unroll=4)
        def vbody(j):
            mw = vb[p, pl.ds(j * 16, 16)]
            for k in range(4):
                s = pl.ds(j * 64 + k * 16, 16)
                eff = (jnp.take_along_axis(mw, gidx + 4 * k, axis=0)
                       >> shv) & 1
                q6 = (qb[p, s] & 63) * 16 + lane
                plsc.addupdate_scatter(tbl, [q6], eff)
                m6 = (mb[p, s] & 63) * 16 + lane + 64 * 16
                plsc.addupdate_scatter(tbl, [m6], eff)
                om = ob[p, s]
                ch = (((om >> 6) ^ om) & 63) * 16 + lane + 128 * 16
                plsc.addupdate_scatter(tbl, [ch], eff)
                f2 = (fb[p, s] & 3) * 16 + lane + 192 * 16
                plsc.addupdate_scatter(tbl, [f2], eff)
        return 0
    lax.fori_loop(0, NCHUNK, chunk_body, 0)

    pltpu.sync_copy(tbl, out_hbm.at[wid])


@functools.cache
def _sc_hist_fn():
    return pl.kernel(
        _sc_hist_body,
        out_type=jax.ShapeDtypeStruct((NW, TBL * 16), jnp.int32),
        mesh=plsc.VectorSubcoreMesh(core_axis_name="c", subcore_axis_name="s"),
        compiler_params=pltpu.CompilerParams(needs_layout_passes=False),
        scratch_types=[
            pltpu.VMEM((2, CH), jnp.int32),
            pltpu.VMEM((2, CH), jnp.int32),
            pltpu.VMEM((2, CH), jnp.int32),
            pltpu.VMEM((2, CH), jnp.int32),
            pltpu.VMEM((2, CH // 4), jnp.int32),
            pltpu.VMEM((TBL * 16,), jnp.int32),
            pltpu.SemaphoreType.DMA((2,)),
        ],
    )


def _sc_hist(*args):
    return _sc_hist_fn()(*args)


def _tc_combined_body(bl_ref, q_ref, f_ref, om_ref, v_ref,
                      qn_ref, fn_ref, on_ref, out_ref):
    q6 = q_ref[...] & 63
    fam = f_ref[...] & 3
    om = om_ref[...]
    ch = ((om >> 6) ^ om) & 63
    eff = v_ref[...]

    is_last = pl.program_id(0) == pl.num_programs(0) - 1
    row = lax.broadcasted_iota(jnp.int32, (R, 128), 0)
    lanei = lax.broadcasted_iota(jnp.int32, (R, 128), 1)
    lastmask = (row == R - 1) & (lanei == 127)

    def nxt(cur, nfirst):
        rolled = pltpu.roll(cur, 127, 1)
        b = jnp.concatenate([cur[1:], nfirst[0:1]], axis=0)
        col0 = jnp.broadcast_to(b[:, 0:1], (R, 128))
        x = jnp.where(lanei == 127, col0, rolled)
        return jnp.where(lastmask & is_last, 0, x)

    q6n = nxt(q6, qn_ref[...] & 63)
    famn = nxt(fam, fn_ref[...] & 3)
    omn = on_ref[...]
    chn = nxt(ch, ((omn >> 6) ^ omn) & 63)

    dq = _pop6(q6 ^ q6n).astype(jnp.float32) * (1.0 / 6.0)
    fx = fam ^ famn
    dfam = (((fx & 1) != 0).astype(jnp.float32)
            + ((fx & 2) != 0).astype(jnp.float32)) * 0.5
    dch = jnp.clip(_pop6(ch ^ chn).astype(jnp.float32) * (1.0 / 6.0), 0.0, 1.0)
    score = jnp.clip(0.5 * dch + 0.35 * dq + 0.15 * dfam, 1e-6, 1.0)
    structural = jnp.where(eff, score, 0.0)
    cosine = jnp.exp(jnp.minimum(bl_ref[...], 0.0))
    out_ref[...] = jnp.clip(0.5 * cosine + 0.5 * structural, 1e-6, 1.0 - 1e-6)


def _tc_combined(bl2, q2, f2, om2, v2):
    blk = pl.BlockSpec((R, 128), lambda i: (i, 0))
    nblk = pl.BlockSpec((8, 128), lambda i: ((i + 1) * (R // 8), 0))
    return pl.pallas_call(
        _tc_combined_body,
        grid=(G,),
        in_specs=[blk, blk, blk, blk, blk, nblk, nblk, nblk],
        out_specs=blk,
        out_shape=jax.ShapeDtypeStruct((ROWS, 128), jnp.float32),
    )(bl2, q2, f2, om2, v2, q2, f2, om2)


def _tc_fold_body(p_ref, q_ref, f_ref, m_ref, s_ref, w_ref, b_ref):
    tot = jnp.sum(p_ref[...], axis=1, keepdims=True)
    q64 = tot[0:64]
    m64 = tot[64:128]
    c64 = tot[128:192]
    q_ref[...] = q64
    m_ref[...] = m64
    f_ref[...] = tot[192:196]

    bi7 = lax.broadcasted_iota(jnp.int32, (64, 7), 0)
    si7 = lax.broadcasted_iota(jnp.int32, (64, 7), 1)
    pop7 = _pop6(bi7)
    qb7 = jnp.broadcast_to(q64, (64, 7))
    cb7 = jnp.broadcast_to(c64, (64, 7))
    s_ref[...] = jnp.sum(jnp.where(pop7 == si7, cb7, 0), axis=0, keepdims=True)
    w_ref[...] = jnp.sum(jnp.where(pop7 == si7, qb7, 0), axis=0, keepdims=True)

    bi6 = lax.broadcasted_iota(jnp.int32, (64, 6), 0)
    si6 = lax.broadcasted_iota(jnp.int32, (64, 6), 1)
    qb6 = jnp.broadcast_to(q64, (64, 6))
    b_ref[...] = jnp.sum(jnp.where(((bi6 >> si6) & 1) != 0, qb6, 0),
                         axis=0, keepdims=True)


def _tc_fold(partial):
    pfold = jnp.transpose(partial.reshape(NW, TBL, 16), (1, 0, 2)).reshape(TBL, NW * 16)
    i32 = jnp.int32
    return pl.pallas_call(
        _tc_fold_body,
        out_shape=(
            jax.ShapeDtypeStruct((64, 1), i32),
            jax.ShapeDtypeStruct((4, 1), i32),
            jax.ShapeDtypeStruct((64, 1), i32),
            jax.ShapeDtypeStruct((1, 7), i32),
            jax.ShapeDtypeStruct((1, 7), i32),
            jax.ShapeDtypeStruct((1, 6), i32),
        ),
    )(pfold)


def kernel(boundary_logprobs, q_class, family, micro_ref, omega12, valid_mask):
    effw = lax.bitcast_convert_type(
        valid_mask.astype(jnp.int8).reshape(N // 4, 4), jnp.int32)

    partial = _sc_hist(q_class, micro_ref, family, omega12, effw)

    bl2 = boundary_logprobs.reshape(ROWS, 128)
    q2 = q_class.reshape(ROWS, 128)
    f2 = family.reshape(ROWS, 128)
    om2 = omega12.reshape(ROWS, 128)
    v2 = valid_mask.reshape(ROWS, 128)
    combined = _tc_combined(bl2, q2, f2, om2, v2).reshape(N)

    qh, fh, mh, sh, wh, bh = _tc_fold(partial)
    return (qh.reshape(64), fh.reshape(4), mh.reshape(64),
            sh.reshape(7), wh.reshape(7), bh.reshape(6), combined)


# speedup vs baseline: 1.0081x; 1.0081x over previous
import functools

import jax
import jax.numpy as jnp
from jax import lax
from jax.experimental import pallas as pl
from jax.experimental.pallas import tpu as pltpu
from jax.experimental.pallas import tpu_sc as plsc

N = 8388608
NW = 32
C = N // NW
CH = 8192
NCHUNK = C // CH
VREGS = CH // 16

TBL = 196

ROWS = N // 128
R = 1024
G = ROWS // R


def _pop6(x):
    return (((x >> 0) & 1) + ((x >> 1) & 1) + ((x >> 2) & 1)
            + ((x >> 3) & 1) + ((x >> 4) & 1) + ((x >> 5) & 1))


def _sc_hist_body(q_hbm, mic_hbm, fam_hbm, om_hbm, eff_hbm, out_hbm,
                  qb, mb, fb, ob, vb, tbl, sem):
    wid = lax.axis_index("s") * 2 + lax.axis_index("c")
    base = wid * C
    lane = lax.iota(jnp.int32, 16)
    zeros16 = jnp.zeros((16,), jnp.int32)
    gidx = lane >> 2
    shv = (lane & 3) * 8

    def zbody(i, _):
        tbl[pl.ds(i * 16, 16)] = zeros16
        return 0
    lax.fori_loop(0, TBL, zbody, 0)

    def issue(c, p):
        off = base + c * CH
        offw = pl.multiple_of((base + c * CH) // 4, 2048)
        pltpu.async_copy(q_hbm.at[pl.ds(off, CH)], qb.at[p], sem.at[p])
        pltpu.async_copy(mic_hbm.at[pl.ds(off, CH)], mb.at[p], sem.at[p])
        pltpu.async_copy(fam_hbm.at[pl.ds(off, CH)], fb.at[p], sem.at[p])
        pltpu.async_copy(om_hbm.at[pl.ds(off, CH)], ob.at[p], sem.at[p])
        pltpu.async_copy(eff_hbm.at[pl.ds(offw, CH // 4)], vb.at[p], sem.at[p])

    def drain(c, p):
        off = base + c * CH
        offw = pl.multiple_of((base + c * CH) // 4, 2048)
        pltpu.make_async_copy(q_hbm.at[pl.ds(off, CH)], qb.at[p], sem.at[p]).wait()
        pltpu.make_async_copy(mic_hbm.at[pl.ds(off, CH)], mb.at[p], sem.at[p]).wait()
        pltpu.make_async_copy(fam_hbm.at[pl.ds(off, CH)], fb.at[p], sem.at[p]).wait()
        pltpu.make_async_copy(om_hbm.at[pl.ds(off, CH)], ob.at[p], sem.at[p]).wait()
        pltpu.make_async_copy(eff_hbm.at[pl.ds(offw, CH // 4)], vb.at[p], sem.at[p]).wait()

    issue(0, 0)

    def chunk_body(c, _):
        p = c & 1

        @pl.when(c + 1 < NCHUNK)
        def _():
            issue(c + 1, 1 - p)

        drain(c, p)

        @plsc.parallel_loop(0, VREGS // 4, unroll=4)
        def vbody(j):
            mw = vb[p, pl.ds(j * 16, 16)]
            for k in range(4):
                s = pl.ds(j * 64 + k * 16, 16)
                eff = (jnp.take_along_axis(mw, gidx + 4 * k, axis=0)
                       >> shv) & 1
                q6 = (qb[p, s] & 63) * 16 + lane
                plsc.addupdate_scatter(tbl, [q6], eff)
                m6 = (mb[p, s] & 63) * 16 + lane + 64 * 16
                plsc.addupdate_scatter(tbl, [m6], eff)
                om = ob[p, s]
                ch = (((om >> 6) ^ om) & 63) * 16 + lane + 128 * 16
                plsc.addupdate_scatter(tbl, [ch], eff)
                f2 = (fb[p, s] & 3) * 16 + lane + 192 * 16
                plsc.addupdate_scatter(tbl, [f2], eff)
        return 0
    lax.fori_loop(0, NCHUNK, chunk_body, 0)

    pltpu.sync_copy(tbl, out_hbm.at[wid])


@functools.cache
def _sc_hist_fn():
    return pl.kernel(
        _sc_hist_body,
        out_type=jax.ShapeDtypeStruct((NW, TBL * 16), jnp.int32),
        mesh=plsc.VectorSubcoreMesh(core_axis_name="c", subcore_axis_name="s"),
        compiler_params=pltpu.CompilerParams(needs_layout_passes=False),
        scratch_types=[
            pltpu.VMEM((2, CH), jnp.int32),
            pltpu.VMEM((2, CH), jnp.int32),
            pltpu.VMEM((2, CH), jnp.int32),
            pltpu.VMEM((2, CH), jnp.int32),
            pltpu.VMEM((2, CH // 4), jnp.int32),
            pltpu.VMEM((TBL * 16,), jnp.int32),
            pltpu.SemaphoreType.DMA((2,)),
        ],
    )


def _sc_hist(*args):
    return _sc_hist_fn()(*args)


def _tc_combined_body(bl_ref, q_ref, f_ref, om_ref, v_ref,
                      qn_ref, fn_ref, on_ref, out_ref):
    q6 = q_ref[...] & 63
    fam = f_ref[...] & 3
    om = om_ref[...]
    ch = ((om >> 6) ^ om) & 63
    eff = v_ref[...] != 0

    is_last = pl.program_id(0) == pl.num_programs(0) - 1
    row = lax.broadcasted_iota(jnp.int32, (R, 128), 0)
    lanei = lax.broadcasted_iota(jnp.int32, (R, 128), 1)
    lastmask = (row == R - 1) & (lanei == 127)

    def nxt(cur, nfirst):
        rolled = pltpu.roll(cur, 127, 1)
        b = jnp.concatenate([cur[1:], nfirst[0:1]], axis=0)
        col0 = jnp.broadcast_to(b[:, 0:1], (R, 128))
        x = jnp.where(lanei == 127, col0, rolled)
        return jnp.where(lastmask & is_last, 0, x)

    q6n = nxt(q6, qn_ref[...] & 63)
    famn = nxt(fam, fn_ref[...] & 3)
    omn = on_ref[...]
    chn = nxt(ch, ((omn >> 6) ^ omn) & 63)

    dq = _pop6(q6 ^ q6n).astype(jnp.float32) * (1.0 / 6.0)
    fx = fam ^ famn
    dfam = (((fx & 1) != 0).astype(jnp.float32)
            + ((fx & 2) != 0).astype(jnp.float32)) * 0.5
    dch = jnp.clip(_pop6(ch ^ chn).astype(jnp.float32) * (1.0 / 6.0), 0.0, 1.0)
    score = jnp.clip(0.5 * dch + 0.35 * dq + 0.15 * dfam, 1e-6, 1.0)
    structural = jnp.where(eff, score, 0.0)
    cosine = jnp.exp(jnp.minimum(bl_ref[...], 0.0))
    out_ref[...] = jnp.clip(0.5 * cosine + 0.5 * structural, 1e-6, 1.0 - 1e-6)


def _tc_combined(bl2, q2, f2, om2, v2):
    blk = pl.BlockSpec((R, 128), lambda i: (i, 0))
    nblk = pl.BlockSpec((8, 128), lambda i: ((i + 1) * (R // 8), 0))
    return pl.pallas_call(
        _tc_combined_body,
        grid=(G,),
        in_specs=[blk, blk, blk, blk, blk, nblk, nblk, nblk],
        out_specs=blk,
        out_shape=jax.ShapeDtypeStruct((ROWS, 128), jnp.float32),
    )(bl2, q2, f2, om2, v2, q2, f2, om2)


def _tc_fold_body(p_ref, q_ref, f_ref, m_ref, s_ref, w_ref, b_ref):
    tot = jnp.sum(p_ref[...], axis=1, keepdims=True)
    q64 = tot[0:64]
    m64 = tot[64:128]
    c64 = tot[128:192]
    q_ref[...] = q64
    m_ref[...] = m64
    f_ref[...] = tot[192:196]

    bi7 = lax.broadcasted_iota(jnp.int32, (64, 7), 0)
    si7 = lax.broadcasted_iota(jnp.int32, (64, 7), 1)
    pop7 = _pop6(bi7)
    qb7 = jnp.broadcast_to(q64, (64, 7))
    cb7 = jnp.broadcast_to(c64, (64, 7))
    s_ref[...] = jnp.sum(jnp.where(pop7 == si7, cb7, 0), axis=0, keepdims=True)
    w_ref[...] = jnp.sum(jnp.where(pop7 == si7, qb7, 0), axis=0, keepdims=True)

    bi6 = lax.broadcasted_iota(jnp.int32, (64, 6), 0)
    si6 = lax.broadcasted_iota(jnp.int32, (64, 6), 1)
    qb6 = jnp.broadcast_to(q64, (64, 6))
    b_ref[...] = jnp.sum(jnp.where(((bi6 >> si6) & 1) != 0, qb6, 0),
                         axis=0, keepdims=True)


def _tc_fold(partial):
    pfold = jnp.transpose(partial.reshape(NW, TBL, 16), (1, 0, 2)).reshape(TBL, NW * 16)
    i32 = jnp.int32
    return pl.pallas_call(
        _tc_fold_body,
        out_shape=(
            jax.ShapeDtypeStruct((64, 1), i32),
            jax.ShapeDtypeStruct((4, 1), i32),
            jax.ShapeDtypeStruct((64, 1), i32),
            jax.ShapeDtypeStruct((1, 7), i32),
            jax.ShapeDtypeStruct((1, 7), i32),
            jax.ShapeDtypeStruct((1, 6), i32),
        ),
    )(pfold)


def kernel(boundary_logprobs, q_class, family, micro_ref, omega12, valid_mask):
    effw = lax.bitcast_convert_type(
        valid_mask.astype(jnp.int8).reshape(N // 4, 4), jnp.int32)

    partial = _sc_hist(q_class, micro_ref, family, omega12, effw)

    bl2 = boundary_logprobs.reshape(ROWS, 128)
    q2 = q_class.reshape(ROWS, 128)
    f2 = family.reshape(ROWS, 128)
    om2 = omega12.reshape(ROWS, 128)
    v2 = valid_mask.astype(jnp.int8).reshape(ROWS, 128)
    combined = _tc_combined(bl2, q2, f2, om2, v2).reshape(N)

    qh, fh, mh, sh, wh, bh = _tc_fold(partial)
    return (qh.reshape(64), fh.reshape(4), mh.reshape(64),
            sh.reshape(7), wh.reshape(7), bh.reshape(6), combined)


# speedup vs baseline: 8.6842x; 8.6145x over previous
import functools

import jax
import jax.numpy as jnp
from jax import lax
from jax.experimental import pallas as pl
from jax.experimental.pallas import tpu as pltpu
from jax.experimental.pallas import tpu_sc as plsc

N = 8388608
NW = 32
C = N // NW
CH = 8192
NCHUNK = C // CH
VREGS = CH // 16

TBL = 196

ROWS = N // 128
R = 1024
G = ROWS // R


def _pop6(x):
    return (((x >> 0) & 1) + ((x >> 1) & 1) + ((x >> 2) & 1)
            + ((x >> 3) & 1) + ((x >> 4) & 1) + ((x >> 5) & 1))


def _sc_hist_body(q_hbm, mic_hbm, fam_hbm, om_hbm, eff_hbm, out_hbm,
                  qb, mb, fb, ob, vb, tbl, sem):
    wid = lax.axis_index("s") * 2 + lax.axis_index("c")
    base = wid * C
    lane = lax.iota(jnp.int32, 16)
    zeros16 = jnp.zeros((16,), jnp.int32)

    def zbody(i, _):
        tbl[pl.ds(i * 16, 16)] = zeros16
        return 0
    lax.fori_loop(0, TBL, zbody, 0)

    def issue(c, p):
        off = base + c * CH
        pltpu.async_copy(q_hbm.at[pl.ds(off, CH)], qb.at[p], sem.at[p])
        pltpu.async_copy(mic_hbm.at[pl.ds(off, CH)], mb.at[p], sem.at[p])
        pltpu.async_copy(fam_hbm.at[pl.ds(off, CH)], fb.at[p], sem.at[p])
        pltpu.async_copy(om_hbm.at[pl.ds(off, CH)], ob.at[p], sem.at[p])
        pltpu.async_copy(eff_hbm.at[pl.ds(off, CH)], vb.at[p], sem.at[p])

    def drain(c, p):
        off = base + c * CH
        pltpu.make_async_copy(q_hbm.at[pl.ds(off, CH)], qb.at[p], sem.at[p]).wait()
        pltpu.make_async_copy(mic_hbm.at[pl.ds(off, CH)], mb.at[p], sem.at[p]).wait()
        pltpu.make_async_copy(fam_hbm.at[pl.ds(off, CH)], fb.at[p], sem.at[p]).wait()
        pltpu.make_async_copy(om_hbm.at[pl.ds(off, CH)], ob.at[p], sem.at[p]).wait()
        pltpu.make_async_copy(eff_hbm.at[pl.ds(off, CH)], vb.at[p], sem.at[p]).wait()

    issue(0, 0)

    def chunk_body(c, _):
        p = c & 1

        @pl.when(c + 1 < NCHUNK)
        def _():
            issue(c + 1, 1 - p)

        drain(c, p)

        @plsc.parallel_loop(0, VREGS, unroll=16)
        def vbody(i):
            s = pl.ds(i * 16, 16)
            eff = vb[p, s]
            q6 = (qb[p, s] & 63) * 16 + lane
            plsc.addupdate_scatter(tbl, [q6], eff)
            m6 = (mb[p, s] & 63) * 16 + lane + 64 * 16
            plsc.addupdate_scatter(tbl, [m6], eff)
            om = ob[p, s]
            ch = (((om >> 6) ^ om) & 63) * 16 + lane + 128 * 16
            plsc.addupdate_scatter(tbl, [ch], eff)
            f2 = (fb[p, s] & 3) * 16 + lane + 192 * 16
            plsc.addupdate_scatter(tbl, [f2], eff)
        return 0
    lax.fori_loop(0, NCHUNK, chunk_body, 0)

    pltpu.sync_copy(tbl, out_hbm.at[wid])


@functools.cache
def _sc_hist_fn():
    return pl.kernel(
        _sc_hist_body,
        out_type=jax.ShapeDtypeStruct((NW, TBL * 16), jnp.int32),
        mesh=plsc.VectorSubcoreMesh(core_axis_name="c", subcore_axis_name="s"),
        compiler_params=pltpu.CompilerParams(needs_layout_passes=False),
        scratch_types=[
            pltpu.VMEM((2, CH), jnp.int32),
            pltpu.VMEM((2, CH), jnp.int32),
            pltpu.VMEM((2, CH), jnp.int32),
            pltpu.VMEM((2, CH), jnp.int32),
            pltpu.VMEM((2, CH), jnp.int32),
            pltpu.VMEM((TBL * 16,), jnp.int32),
            pltpu.SemaphoreType.DMA((2,)),
        ],
    )


def _sc_hist(*args):
    return _sc_hist_fn()(*args)


def _tc_combined_body(bl_ref, q_ref, f_ref, om_ref, v_ref,
                      qn_ref, fn_ref, on_ref, out_ref):
    q6 = q_ref[...] & 63
    fam = f_ref[...] & 3
    om = om_ref[...]
    ch = ((om >> 6) ^ om) & 63
    eff = v_ref[...] != 0

    is_last = pl.program_id(0) == pl.num_programs(0) - 1
    row = lax.broadcasted_iota(jnp.int32, (R, 128), 0)
    lanei = lax.broadcasted_iota(jnp.int32, (R, 128), 1)
    lastmask = (row == R - 1) & (lanei == 127)

    def nxt(cur, nfirst):
        rolled = pltpu.roll(cur, 127, 1)
        b = jnp.concatenate([cur[1:], nfirst[0:1]], axis=0)
        col0 = jnp.broadcast_to(b[:, 0:1], (R, 128))
        x = jnp.where(lanei == 127, col0, rolled)
        return jnp.where(lastmask & is_last, 0, x)

    q6n = nxt(q6, qn_ref[...] & 63)
    famn = nxt(fam, fn_ref[...] & 3)
    omn = on_ref[...]
    chn = nxt(ch, ((omn >> 6) ^ omn) & 63)

    dq = _pop6(q6 ^ q6n).astype(jnp.float32) * (1.0 / 6.0)
    fx = fam ^ famn
    dfam = (((fx & 1) != 0).astype(jnp.float32)
            + ((fx & 2) != 0).astype(jnp.float32)) * 0.5
    dch = jnp.clip(_pop6(ch ^ chn).astype(jnp.float32) * (1.0 / 6.0), 0.0, 1.0)
    score = jnp.clip(0.5 * dch + 0.35 * dq + 0.15 * dfam, 1e-6, 1.0)
    structural = jnp.where(eff, score, 0.0)
    cosine = jnp.exp(jnp.minimum(bl_ref[...], 0.0))
    out_ref[...] = jnp.clip(0.5 * cosine + 0.5 * structural, 1e-6, 1.0 - 1e-6)


def _tc_combined(bl2, q2, f2, om2, v2):
    blk = pl.BlockSpec((R, 128), lambda i: (i, 0))
    nblk = pl.BlockSpec((8, 128), lambda i: ((i + 1) * (R // 8), 0))
    return pl.pallas_call(
        _tc_combined_body,
        grid=(G,),
        in_specs=[blk, blk, blk, blk, blk, nblk, nblk, nblk],
        out_specs=blk,
        out_shape=jax.ShapeDtypeStruct((ROWS, 128), jnp.float32),
    )(bl2, q2, f2, om2, v2, q2, f2, om2)


def _tc_fold_body(p_ref, q_ref, f_ref, m_ref, s_ref, w_ref, b_ref):
    tot = jnp.sum(p_ref[...], axis=1, keepdims=True)
    q64 = tot[0:64]
    m64 = tot[64:128]
    c64 = tot[128:192]
    q_ref[...] = q64
    m_ref[...] = m64
    f_ref[...] = tot[192:196]

    bi7 = lax.broadcasted_iota(jnp.int32, (64, 7), 0)
    si7 = lax.broadcasted_iota(jnp.int32, (64, 7), 1)
    pop7 = _pop6(bi7)
    qb7 = jnp.broadcast_to(q64, (64, 7))
    cb7 = jnp.broadcast_to(c64, (64, 7))
    s_ref[...] = jnp.sum(jnp.where(pop7 == si7, cb7, 0), axis=0, keepdims=True)
    w_ref[...] = jnp.sum(jnp.where(pop7 == si7, qb7, 0), axis=0, keepdims=True)

    bi6 = lax.broadcasted_iota(jnp.int32, (64, 6), 0)
    si6 = lax.broadcasted_iota(jnp.int32, (64, 6), 1)
    qb6 = jnp.broadcast_to(q64, (64, 6))
    b_ref[...] = jnp.sum(jnp.where(((bi6 >> si6) & 1) != 0, qb6, 0),
                         axis=0, keepdims=True)


def _tc_fold(partial):
    pfold = jnp.transpose(partial.reshape(NW, TBL, 16), (1, 0, 2)).reshape(TBL, NW * 16)
    i32 = jnp.int32
    return pl.pallas_call(
        _tc_fold_body,
        out_shape=(
            jax.ShapeDtypeStruct((64, 1), i32),
            jax.ShapeDtypeStruct((4, 1), i32),
            jax.ShapeDtypeStruct((64, 1), i32),
            jax.ShapeDtypeStruct((1, 7), i32),
            jax.ShapeDtypeStruct((1, 7), i32),
            jax.ShapeDtypeStruct((1, 6), i32),
        ),
    )(pfold)


def kernel(boundary_logprobs, q_class, family, micro_ref, omega12, valid_mask):
    eff32 = valid_mask.astype(jnp.int32)

    partial = _sc_hist(q_class, micro_ref, family, omega12, eff32)

    bl2 = boundary_logprobs.reshape(ROWS, 128)
    q2 = q_class.reshape(ROWS, 128)
    f2 = family.reshape(ROWS, 128)
    om2 = omega12.reshape(ROWS, 128)
    v2 = valid_mask.astype(jnp.int8).reshape(ROWS, 128)
    combined = _tc_combined(bl2, q2, f2, om2, v2).reshape(N)

    qh, fh, mh, sh, wh, bh = _tc_fold(partial)
    return (qh.reshape(64), fh.reshape(4), mh.reshape(64),
            sh.reshape(7), wh.reshape(7), bh.reshape(6), combined)
